# baseline (speedup 1.0000x reference)
import functools
import math

import jax
import jax.numpy as jnp
import numpy as np
from jax.experimental import pallas as pl

_NUM_ANGLE = 100
_NUM_RHO = 100
_A_BLK = 5


@functools.lru_cache(maxsize=None)
def _onehot_table(H, W):
    irho = int(math.sqrt(H * H + W * W) + 1) / float(_NUM_RHO)
    itheta = math.pi / _NUM_ANGLE
    half = _NUM_ANGLE // 2
    angles = np.arange(half, dtype=np.float64) * itheta
    cosv = (np.cos(angles) / irho).astype(np.float32)
    sinv = (np.sin(angles) / irho).astype(np.float32)
    ys, xs = np.meshgrid(np.arange(H + 1), np.arange(W), indexing="ij")
    xx = (xs - W // 2).reshape(-1).astype(np.float32)
    yy = (ys - H // 2).reshape(-1).astype(np.float32)
    r = np.round(xx[None, :] * cosv[:, None] + yy[None, :] * sinv[:, None])
    r = r.astype(np.int32) + _NUM_RHO // 2
    r = np.clip(r, 0, _NUM_RHO - 1)
    onehot = r[:, None, :] == np.arange(_NUM_RHO, dtype=np.int32)[None, :, None]
    nblk = half // _A_BLK
    onehot = onehot.reshape(nblk, _A_BLK * _NUM_RHO, (H + 1) * W)
    return onehot.astype(jnp.bfloat16)


def _dht_body(onehot_ref, x_ref, out_ref):
    out_ref[0] = jax.lax.dot_general(
        x_ref[...],
        onehot_ref[0],
        dimension_numbers=(((1,), (1,)), ((), ())),
        preferred_element_type=jnp.float32,
    )


def kernel(x):
    B, C, H, W = x.shape
    BC = B * C
    HWE = (H + 1) * W
    half = _NUM_ANGLE // 2
    nblk = half // _A_BLK
    onehot = jnp.asarray(_onehot_table(H, W))

    ximg = x.reshape(BC, H, W).astype(jnp.bfloat16)
    xlo = jnp.pad(ximg, ((0, 0), (0, 1), (0, 0)))
    xrot = jnp.pad(
        jnp.transpose(ximg, (0, 2, 1))[:, ::-1, :], ((0, 0), (1, 0), (0, 0))
    )
    xcat = jnp.concatenate([xlo, xrot], axis=0).reshape(2 * BC, HWE)

    out = pl.pallas_call(
        _dht_body,
        grid=(nblk,),
        in_specs=[
            pl.BlockSpec((1, _A_BLK * _NUM_RHO, HWE), lambda i: (i, 0, 0)),
            pl.BlockSpec((2 * BC, HWE), lambda i: (0, 0)),
        ],
        out_specs=pl.BlockSpec((1, 2 * BC, _A_BLK * _NUM_RHO), lambda i: (i, 0, 0)),
        out_shape=jax.ShapeDtypeStruct(
            (nblk, 2 * BC, _A_BLK * _NUM_RHO), jnp.float32
        ),
    )(onehot, xcat)

    out = out.reshape(nblk, 2 * BC, _A_BLK, _NUM_RHO)
    lo = jnp.transpose(out[:, :BC], (1, 0, 2, 3)).reshape(BC, half, _NUM_RHO)
    hi = jnp.transpose(out[:, BC:], (1, 0, 2, 3)).reshape(BC, half, _NUM_RHO)
    acc = jnp.concatenate([lo, hi], axis=1)
    return acc.reshape(B, C, _NUM_ANGLE, _NUM_RHO)
